# baseline (speedup 1.0000x reference)
import functools

import jax
import jax.numpy as jnp
from jax import lax
from jax.experimental import pallas as pl
from jax.experimental.pallas import tpu as pltpu
from jax.experimental.pallas import tpu_sc as plsc

N = 160000
D = 512
H = 128
NUM_SEG = 10000

ROWS = N // 128


BR = 6400
NBLK = N // BR
SPLIT_BLK = 20
SPLIT = SPLIT_BLK * BR // 128


def _mlp_body(x_ref, w1_ref, b1_ref, w2_ref, b2_ref, o_ref):
    h_t = jax.lax.dot_general(
        w1_ref[...], x_ref[...],
        dimension_numbers=(((0,), (1,)), ((), ())),
        preferred_element_type=jnp.float32,
    )
    h_t = jnp.maximum(h_t + b1_ref[...], 0.0)
    p = jnp.sum(h_t * w2_ref[...], axis=0)
    o_ref[...] = p.reshape(1, 1, -1) + b2_ref[0]


def _mlp_part(embeds, W1, b1, W2, b2, blk0, nblk):
    return pl.pallas_call(
        _mlp_body,
        grid=(nblk,),
        in_specs=[
            pl.BlockSpec((BR, D), lambda i: (i + blk0, 0)),
            pl.BlockSpec((D, H), lambda i: (0, 0)),
            pl.BlockSpec((H, 1), lambda i: (0, 0)),
            pl.BlockSpec((H, 1), lambda i: (0, 0)),
            pl.BlockSpec(memory_space=pltpu.SMEM),
        ],
        out_specs=pl.BlockSpec((1, 1, BR), lambda i: (i, 0, 0)),
        out_shape=jax.ShapeDtypeStruct((nblk, 1, BR), jnp.float32),
        compiler_params=pltpu.CompilerParams(
            dimension_semantics=("parallel",),
        ),
    )(embeds, W1, b1.reshape(H, 1), W2, b2)


NS = 16
ACC = 10240
SLICE = ACC // NS
STEP = 8


def _make_segsum(core_cfgs, merge):
    ncores = len(core_cfgs)
    maxrows = max(max(t, l + tl) for (_, _, t, _, l, tl) in core_cfgs)

    def body(probs_hbm, idx_hbm, probs_t_hbm, idx_t_hbm, init_hbm, out_hbm,
             idx_v, probs_v, out_v, tmp_v, acc_sh, sem):
        cid = lax.axis_index("c")
        sid = lax.axis_index("s")

        def staged(cfg):
            irow0, prow0, trows, full, last, tail = cfg

            @pl.when(sid < full)
            def _():
                gb = pl.multiple_of(irow0 + sid * trows, 8)
                pb_ = pl.multiple_of(prow0 + sid * trows, 8)
                pltpu.sync_copy(idx_hbm.at[pl.ds(gb, trows)],
                                idx_v.at[pl.ds(0, trows)])
                pltpu.sync_copy(probs_hbm.at[pl.ds(pb_, trows)],
                                probs_v.at[pl.ds(0, trows)])

            if last or tail:
                @pl.when(sid == NS - 1)
                def _():
                    if last:
                        pltpu.sync_copy(
                            idx_hbm.at[pl.ds(irow0 + full * trows, last)],
                            idx_v.at[pl.ds(0, last)])
                        pltpu.sync_copy(
                            probs_hbm.at[pl.ds(prow0 + full * trows, last)],
                            probs_v.at[pl.ds(0, last)])
                    if tail:
                        pltpu.sync_copy(idx_t_hbm, idx_v.at[pl.ds(last, tail)])
                        pltpu.sync_copy(probs_t_hbm,
                                        probs_v.at[pl.ds(last, tail)])

        def scattered(cfg):
            _, _, trows, full, last, tail = cfg

            def scatter_rows(nrows):
                cps = []
                for c in range(nrows):
                    if c >= STEP:
                        cps[c - STEP].wait()
                    cps.append(pltpu.async_copy(
                        probs_v.at[c], acc_sh.at[idx_v.at[c]], sem, add=True
                    ))
                for cp in cps[max(0, nrows - STEP):]:
                    cp.wait()

            @pl.when(sid < full)
            def _():
                scatter_rows(trows)

            if last or tail:
                @pl.when(sid == NS - 1)
                def _():
                    scatter_rows(last + tail)

        for ci, cfg in enumerate(core_cfgs):
            if ncores == 1:
                staged(cfg)
            else:
                @pl.when(cid == ci)
                def _(cfg=cfg):
                    staged(cfg)

        @pl.when(sid == 0)
        def _():
            pltpu.sync_copy(init_hbm.at[cid], acc_sh)

        plsc.subcore_barrier()

        for ci, cfg in enumerate(core_cfgs):
            if ncores == 1:
                scattered(cfg)
            else:
                @pl.when(cid == ci)
                def _(cfg=cfg):
                    scattered(cfg)

        plsc.subcore_barrier()

        off = pl.multiple_of(sid * SLICE, SLICE)
        pltpu.sync_copy(acc_sh.at[pl.ds(off, SLICE)], out_v)
        if merge:
            pltpu.sync_copy(init_hbm.at[1].at[pl.ds(off, SLICE)], tmp_v)
            for i in range(SLICE // 16):
                s = pl.ds(i * 16, 16)
                out_v[s] = out_v[s] + tmp_v[s]
        pltpu.sync_copy(out_v, out_hbm.at[cid, sid])

    mesh = plsc.VectorSubcoreMesh(
        core_axis_name="c", subcore_axis_name="s", num_cores=ncores
    )
    return functools.partial(
        pl.kernel,
        mesh=mesh,
        out_type=jax.ShapeDtypeStruct((ncores, NS, SLICE), jnp.float32),
        scratch_types=[
            pltpu.VMEM((maxrows, 128), jnp.int32),
            pltpu.VMEM((maxrows, 128), jnp.float32),
            pltpu.VMEM((SLICE,), jnp.float32),
            pltpu.VMEM((SLICE,), jnp.float32),
            pltpu.VMEM_SHARED((ACC,), jnp.float32),
            pltpu.SemaphoreType.DMA,
        ],
    )(body)


_SEG_A = _make_segsum(
    [(0, 0, 32, 16, 0, 0), (512, 512, 32, 15, 8, 0)], merge=False)
_SEG_B = _make_segsum(
    [(SPLIT, 0, 16, 15, 8, 2)], merge=True)


def kernel(embeds, indices, W1, b1, W2, b2):
    idx2d = indices.astype(jnp.int32).reshape(ROWS, 128)
    zeros = jnp.zeros((2, ACC), jnp.float32)

    pa = _mlp_part(embeds, W1, b1, W2, b2, 0, SPLIT_BLK).reshape(SPLIT, 128)
    pb = _mlp_part(embeds, W1, b1, W2, b2, SPLIT_BLK, NBLK - SPLIT_BLK)
    pb = pb.reshape(ROWS - SPLIT, 128)

    sa = _SEG_A(pa, idx2d, pa[:2], idx2d[:2], zeros)
    sb = _SEG_B(pb, idx2d, pb[-2:], idx2d[-2:], sa.reshape(2, ACC))
    return sb.reshape(-1)[:NUM_SEG]


# speedup vs baseline: 1.0162x; 1.0162x over previous
import functools

import jax
import jax.numpy as jnp
from jax import lax
from jax.experimental import pallas as pl
from jax.experimental.pallas import tpu as pltpu
from jax.experimental.pallas import tpu_sc as plsc

N = 160000
D = 512
H = 128
NUM_SEG = 10000

ROWS = N // 128


BR = 6400
NBLK = N // BR
SPLIT_BLK = 20
SPLIT = SPLIT_BLK * BR // 128


def _mlp_body(x_ref, w1_ref, b1_ref, w2_ref, b2_ref, o_ref):
    h_t = jax.lax.dot_general(
        w1_ref[...], x_ref[...],
        dimension_numbers=(((0,), (1,)), ((), ())),
        preferred_element_type=jnp.float32,
    )
    h_t = jnp.maximum(h_t + b1_ref[...], 0.0)
    p = jnp.sum(h_t * w2_ref[...], axis=0)
    o_ref[...] = p.reshape(1, 1, -1) + b2_ref[0]


def _mlp_part(embeds, W1, b1, W2, b2, blk0, nblk):
    return pl.pallas_call(
        _mlp_body,
        grid=(nblk,),
        in_specs=[
            pl.BlockSpec((BR, D), lambda i: (i + blk0, 0)),
            pl.BlockSpec((D, H), lambda i: (0, 0)),
            pl.BlockSpec((H, 1), lambda i: (0, 0)),
            pl.BlockSpec((H, 1), lambda i: (0, 0)),
            pl.BlockSpec(memory_space=pltpu.SMEM),
        ],
        out_specs=pl.BlockSpec((1, 1, BR), lambda i: (i, 0, 0)),
        out_shape=jax.ShapeDtypeStruct((nblk, 1, BR), jnp.float32),
        compiler_params=pltpu.CompilerParams(
            dimension_semantics=("parallel",),
        ),
    )(embeds, W1, b1.reshape(H, 1), W2, b2)


NS = 16
ACC = 10240
SLICE = ACC // NS
STEP = 8


def _make_segsum(core_cfgs, merge):
    ncores = len(core_cfgs)
    maxrows = max(max(t, l + tl) for (_, _, t, _, l, tl) in core_cfgs)

    def body(probs_hbm, idx_hbm, probs_t_hbm, idx_t_hbm, init_hbm, out_hbm,
             idx_v, probs_v, out_v, tmp_v, acc_sh, sem):
        cid = lax.axis_index("c")
        sid = lax.axis_index("s")

        def staged(cfg):
            irow0, prow0, trows, full, last, tail = cfg

            @pl.when(sid < full)
            def _():
                gb = pl.multiple_of(irow0 + sid * trows, 8)
                pb_ = pl.multiple_of(prow0 + sid * trows, 8)
                pltpu.sync_copy(idx_hbm.at[pl.ds(gb, trows)],
                                idx_v.at[pl.ds(0, trows)])
                pltpu.sync_copy(probs_hbm.at[pl.ds(pb_, trows)],
                                probs_v.at[pl.ds(0, trows)])

            if last or tail:
                @pl.when(sid == NS - 1)
                def _():
                    if last:
                        pltpu.sync_copy(
                            idx_hbm.at[pl.ds(irow0 + full * trows, last)],
                            idx_v.at[pl.ds(0, last)])
                        pltpu.sync_copy(
                            probs_hbm.at[pl.ds(prow0 + full * trows, last)],
                            probs_v.at[pl.ds(0, last)])
                    if tail:
                        pltpu.sync_copy(idx_t_hbm, idx_v.at[pl.ds(last, tail)])
                        pltpu.sync_copy(probs_t_hbm,
                                        probs_v.at[pl.ds(last, tail)])

        def scattered(cfg):
            _, _, trows, full, last, tail = cfg

            def scatter_rows(nrows):
                cps = []
                for c in range(nrows):
                    if c >= STEP:
                        cps[c - STEP].wait()
                    cps.append(pltpu.async_copy(
                        probs_v.at[c], acc_sh.at[idx_v.at[c]], sem, add=True
                    ))
                for cp in cps[max(0, nrows - STEP):]:
                    cp.wait()

            @pl.when(sid < full)
            def _():
                scatter_rows(trows)

            if last or tail:
                @pl.when(sid == NS - 1)
                def _():
                    scatter_rows(last + tail)

        for ci, cfg in enumerate(core_cfgs):
            if ncores == 1:
                staged(cfg)
            else:
                @pl.when(cid == ci)
                def _(cfg=cfg):
                    staged(cfg)

        @pl.when(sid == 0)
        def _():
            pltpu.sync_copy(init_hbm.at[cid], acc_sh)

        plsc.subcore_barrier()

        for ci, cfg in enumerate(core_cfgs):
            if ncores == 1:
                scattered(cfg)
            else:
                @pl.when(cid == ci)
                def _(cfg=cfg):
                    scattered(cfg)

        plsc.subcore_barrier()

        off = pl.multiple_of(sid * SLICE, SLICE)
        pltpu.sync_copy(acc_sh.at[pl.ds(off, SLICE)], out_v)
        if merge:
            pltpu.sync_copy(init_hbm.at[1].at[pl.ds(off, SLICE)], tmp_v)
            for i in range(SLICE // 16):
                s = pl.ds(i * 16, 16)
                out_v[s] = out_v[s] + tmp_v[s]
        pltpu.sync_copy(out_v, out_hbm.at[cid, sid])

    mesh = plsc.VectorSubcoreMesh(
        core_axis_name="c", subcore_axis_name="s", num_cores=ncores
    )
    return functools.partial(
        pl.kernel,
        mesh=mesh,
        out_type=jax.ShapeDtypeStruct((ncores, NS, SLICE), jnp.float32),
        scratch_types=[
            pltpu.VMEM((maxrows, 128), jnp.int32),
            pltpu.VMEM((maxrows, 128), jnp.float32),
            pltpu.VMEM((SLICE,), jnp.float32),
            pltpu.VMEM((SLICE,), jnp.float32),
            pltpu.VMEM_SHARED((ACC,), jnp.float32),
            pltpu.SemaphoreType.DMA,
        ],
    )(body)


_SEG_A = _make_segsum([(0, 0, 64, 15, 40, 0)], merge=False)
_SEG_B = _make_segsum([(SPLIT, 0, 16, 15, 8, 2)], merge=False)


def kernel(embeds, indices, W1, b1, W2, b2):
    idx2d = indices.astype(jnp.int32).reshape(ROWS, 128)
    zeros = jnp.zeros((1, ACC), jnp.float32)

    pa = _mlp_part(embeds, W1, b1, W2, b2, 0, SPLIT_BLK).reshape(SPLIT, 128)
    pb = _mlp_part(embeds, W1, b1, W2, b2, SPLIT_BLK, NBLK - SPLIT_BLK)
    pb = pb.reshape(ROWS - SPLIT, 128)

    sa = _SEG_A(pa, idx2d, pa[:2], idx2d[:2], zeros)
    sb = _SEG_B(pb, idx2d, pb[-2:], idx2d[-2:], sa.reshape(1, ACC))
    return sb.reshape(-1)[:NUM_SEG]


# speedup vs baseline: 1.0235x; 1.0071x over previous
import functools

import jax
import jax.numpy as jnp
from jax import lax
from jax.experimental import pallas as pl
from jax.experimental.pallas import tpu as pltpu
from jax.experimental.pallas import tpu_sc as plsc

N = 160000
D = 512
H = 128
NUM_SEG = 10000

ROWS = N // 128


BR = 6400
NBLK = N // BR
SPLIT_BLK = 20
SPLIT = SPLIT_BLK * BR // 128


def _mlp_body(x_ref, w1_ref, b1_ref, w2_ref, b2_ref, o_ref):
    h_t = jax.lax.dot_general(
        w1_ref[...], x_ref[...],
        dimension_numbers=(((0,), (1,)), ((), ())),
        preferred_element_type=jnp.float32,
    )
    h_t = jnp.maximum(h_t + b1_ref[...], 0.0)
    p = jnp.sum(h_t * w2_ref[...], axis=0)
    o_ref[...] = p.reshape(1, 1, -1) + b2_ref[0]


def _mlp_part(embeds, W1, b1, W2, b2, blk0, nblk):
    return pl.pallas_call(
        _mlp_body,
        grid=(nblk,),
        in_specs=[
            pl.BlockSpec((BR, D), lambda i: (i + blk0, 0)),
            pl.BlockSpec((D, H), lambda i: (0, 0)),
            pl.BlockSpec((H, 1), lambda i: (0, 0)),
            pl.BlockSpec((H, 1), lambda i: (0, 0)),
            pl.BlockSpec(memory_space=pltpu.SMEM),
        ],
        out_specs=pl.BlockSpec((1, 1, BR), lambda i: (i, 0, 0)),
        out_shape=jax.ShapeDtypeStruct((nblk, 1, BR), jnp.float32),
        compiler_params=pltpu.CompilerParams(
            dimension_semantics=("parallel",),
        ),
    )(embeds, W1, b1.reshape(H, 1), W2, b2)


NS = 16
ACC = 10240
SLICE = ACC // NS
STEP = 8


def _plan_tile(elem0, n):
    out, pos = [], 0
    while n > 0:
        j, o = divmod(elem0, BR)
        ln = min(BR - o, n)
        out.append((j, o, ln, pos))
        elem0 += ln
        pos += ln
        n -= ln
    return out


def _make_segsum(idx_cfg, probs_plans):
    irow0, trows, full, last, tail = idx_cfg
    nrows_t = [len_ // 128 for len_ in
               [sum(ln for (_, _, ln, _) in p) for p in probs_plans]]

    def body(probs_hbm, idx_hbm, idx_t_hbm, init_hbm, out_hbm,
             idx_v, probs_v, out_v, acc_sh, sem):
        sid = lax.axis_index("s")

        for t, plan in enumerate(probs_plans):
            if not plan:
                continue

            @pl.when(sid == t)
            def _(plan=plan):
                for (j, o, ln, pos) in plan:
                    pltpu.sync_copy(probs_hbm.at[j, 0, pl.ds(o, ln)],
                                    probs_v.at[pl.ds(pos, ln)])

        @pl.when(sid < full)
        def _():
            gb = pl.multiple_of(irow0 + sid * trows, 8)
            pltpu.sync_copy(idx_hbm.at[pl.ds(gb, trows)],
                            idx_v.at[pl.ds(0, trows)])

        if last or tail:
            @pl.when(sid == NS - 1)
            def _():
                if last:
                    pltpu.sync_copy(
                        idx_hbm.at[pl.ds(irow0 + full * trows, last)],
                        idx_v.at[pl.ds(0, last)])
                if tail:
                    pltpu.sync_copy(idx_t_hbm, idx_v.at[pl.ds(last, tail)])

        @pl.when(sid == 0)
        def _():
            pltpu.sync_copy(init_hbm, acc_sh)

        plsc.subcore_barrier()

        def scatter_rows(nrows):
            cps = []
            for c in range(nrows):
                if c >= STEP:
                    cps[c - STEP].wait()
                cps.append(pltpu.async_copy(
                    probs_v.at[pl.ds(c * 128, 128)],
                    acc_sh.at[idx_v.at[c]],
                    sem,
                    add=True,
                ))
            for cp in cps[max(0, nrows - STEP):]:
                cp.wait()

        for t, nr in enumerate(nrows_t):
            if not nr:
                continue

            @pl.when(sid == t)
            def _(nr=nr):
                scatter_rows(nr)

        plsc.subcore_barrier()

        off = pl.multiple_of(sid * SLICE, SLICE)
        pltpu.sync_copy(acc_sh.at[pl.ds(off, SLICE)], out_v)
        pltpu.sync_copy(out_v, out_hbm.at[sid])

    mesh = plsc.VectorSubcoreMesh(
        core_axis_name="c", subcore_axis_name="s", num_cores=1
    )
    maxwords = max(sum(ln for (_, _, ln, _) in p) for p in probs_plans)
    maxrows = max(max(trows, last + tail), maxwords // 128)
    return functools.partial(
        pl.kernel,
        mesh=mesh,
        out_type=jax.ShapeDtypeStruct((NS, SLICE), jnp.float32),
        scratch_types=[
            pltpu.VMEM((maxrows, 128), jnp.int32),
            pltpu.VMEM((maxrows * 128,), jnp.float32),
            pltpu.VMEM((SLICE,), jnp.float32),
            pltpu.VMEM_SHARED((ACC,), jnp.float32),
            pltpu.SemaphoreType.DMA,
        ],
    )(body)


_PLANS_A = [_plan_tile(8192 * t, 8192) for t in range(15)]
_PLANS_A.append(_plan_tile(8192 * 15, 5120))
_SEG_A = _make_segsum((0, 64, 15, 40, 0), _PLANS_A)

_PLANS_B = [_plan_tile(2048 * t, 2048) for t in range(15)]
_PLANS_B.append(_plan_tile(2048 * 15, 1280))
_SEG_B = _make_segsum((SPLIT, 16, 15, 8, 2), _PLANS_B)


def kernel(embeds, indices, W1, b1, W2, b2):
    idx2d = indices.astype(jnp.int32).reshape(ROWS, 128)
    zeros = jnp.zeros((ACC,), jnp.float32)

    pa = _mlp_part(embeds, W1, b1, W2, b2, 0, SPLIT_BLK)
    pb = _mlp_part(embeds, W1, b1, W2, b2, SPLIT_BLK, NBLK - SPLIT_BLK)

    sa = _SEG_A(pa, idx2d, idx2d[:2], zeros)
    sb = _SEG_B(pb, idx2d, idx2d[-2:], sa.reshape(-1))
    return sb.reshape(-1)[:NUM_SEG]
